# baseline (speedup 1.0000x reference)
import math

import jax
import jax.numpy as jnp
from jax import lax
from jax.experimental import pallas as pl
from jax.experimental.pallas import tpu as pltpu
from jax.experimental.pallas import tpu_sc as plsc

_CUTOFF = 0.5
_E_PARENT = 6_400_000
_E_FILTER = 3_200_000

_K = 8000
_NCHUNKS = _E_FILTER // _K
_NW = 32
_JMAX = -(-_NCHUNKS // _NW)

_S3 = -1.0 / 6.0
_S5 = 1.0 / 120.0
_S7 = -1.0 / 5040.0
_S9 = 1.0 / 362880.0


def _dist_body(dist_hbm, idx_hbm, distf_hbm, sw_hbm, mask_hbm,
               idx_v, dist_v, sw_v, mask_v, sem_d):
    wid = lax.axis_index("s") * 2 + lax.axis_index("c")
    ones_i = jnp.ones((16,), jnp.int32)
    zeros_i = jnp.zeros((16,), jnp.int32)

    def chunk_body(j, carry):
        c = wid + _NW * j

        @pl.when(c < _NCHUNKS)
        def _():
            base = c * _K
            pltpu.sync_copy(idx_hbm.at[pl.ds(base, _K)], idx_v)
            cp_d = pltpu.async_copy(dist_hbm.at[idx_v], dist_v, sem_d)
            cp_d.wait()

            def group(g, carry):
                o = g * 16
                d = dist_v[pl.ds(o, 16)]
                m = d < _CUTOFF
                z = d * (math.pi / _CUTOFF) - (math.pi / 2.0)
                z2 = z * z
                s = z * (1.0 + z2 * (_S3 + z2 * (_S5 + z2 * (_S7 + z2 * _S9))))
                sw_v[pl.ds(o, 16)] = jnp.where(m, 0.5 - 0.5 * s, 0.0)
                mask_v[pl.ds(o, 16)] = jnp.where(m, ones_i, zeros_i)
                return carry

            lax.fori_loop(0, _K // 16, group, 0, unroll=4)

            pltpu.sync_copy(dist_v, distf_hbm.at[pl.ds(base, _K)])
            pltpu.sync_copy(sw_v, sw_hbm.at[pl.ds(base, _K)])
            pltpu.sync_copy(mask_v, mask_hbm.at[pl.ds(base, _K)])

        return carry

    lax.fori_loop(0, _JMAX, chunk_body, 0)


def _vec_body(vx_hbm, vy_hbm, vz_hbm, idx_hbm, ox_hbm, oy_hbm, oz_hbm,
              idx_v, px_v, py_v, pz_v, sem_v):
    wid = lax.axis_index("s") * 2 + lax.axis_index("c")

    def chunk_body(j, carry):
        c = wid + _NW * j

        @pl.when(c < _NCHUNKS)
        def _():
            base = c * _K
            pltpu.sync_copy(idx_hbm.at[pl.ds(base, _K)], idx_v)
            cp_x = pltpu.async_copy(vx_hbm.at[idx_v], px_v, sem_v)
            cp_y = pltpu.async_copy(vy_hbm.at[idx_v], py_v, sem_v)
            cp_z = pltpu.async_copy(vz_hbm.at[idx_v], pz_v, sem_v)
            cp_x.wait()
            cp_y.wait()
            cp_z.wait()
            pltpu.sync_copy(px_v, ox_hbm.at[pl.ds(base, _K)])
            pltpu.sync_copy(py_v, oy_hbm.at[pl.ds(base, _K)])
            pltpu.sync_copy(pz_v, oz_hbm.at[pl.ds(base, _K)])

        return carry

    lax.fori_loop(0, _JMAX, chunk_body, 0)


@jax.jit
def _run(vec, distances, filter_indices):
    mesh = plsc.VectorSubcoreMesh(core_axis_name="c", subcore_axis_name="s")
    f32 = jnp.float32
    dist_fn = pl.kernel(
        _dist_body,
        out_type=[
            jax.ShapeDtypeStruct((_E_FILTER,), f32),
            jax.ShapeDtypeStruct((_E_FILTER,), f32),
            jax.ShapeDtypeStruct((_E_FILTER,), jnp.int32),
        ],
        mesh=mesh,
        scratch_types=[
            pltpu.VMEM((_K,), jnp.int32),
            pltpu.VMEM((_K,), f32),
            pltpu.VMEM((_K,), f32),
            pltpu.VMEM((_K,), jnp.int32),
            pltpu.SemaphoreType.DMA,
        ],
    )
    vec_fn = pl.kernel(
        _vec_body,
        out_type=[
            jax.ShapeDtypeStruct((_E_FILTER,), f32),
            jax.ShapeDtypeStruct((_E_FILTER,), f32),
            jax.ShapeDtypeStruct((_E_FILTER,), f32),
        ],
        mesh=mesh,
        scratch_types=[
            pltpu.VMEM((_K,), jnp.int32),
            pltpu.VMEM((_K,), f32),
            pltpu.VMEM((_K,), f32),
            pltpu.VMEM((_K,), f32),
            pltpu.SemaphoreType.DMA,
        ],
    )
    dist_f, switch, mask_i32 = dist_fn(distances, filter_indices)
    ox, oy, oz = vec_fn(vec[:, 0], vec[:, 1], vec[:, 2], filter_indices)
    vec_f = jnp.stack([ox, oy, oz], axis=1)
    return vec_f, dist_f, switch, mask_i32.astype(jnp.bool_)


def kernel(vec, distances, filter_indices):
    return _run(vec, distances, filter_indices)


# speedup vs baseline: 1.1570x; 1.1570x over previous
import math

import jax
import jax.numpy as jnp
from jax import lax
from jax.experimental import pallas as pl
from jax.experimental.pallas import tpu as pltpu
from jax.experimental.pallas import tpu_sc as plsc

_CUTOFF = 0.5
_E_PARENT = 6_400_000
_E_FILTER = 3_200_000

_K = 8000
_NCHUNKS = _E_FILTER // _K
_NW = 32
_JMAX = -(-_NCHUNKS // _NW)

_S3 = -1.0 / 6.0
_S5 = 1.0 / 120.0
_S7 = -1.0 / 5040.0
_S9 = 1.0 / 362880.0


def _body(vx_hbm, vy_hbm, vz_hbm, dist_hbm, idx_hbm,
          ox_hbm, oy_hbm, oz_hbm, distf_hbm, sw_hbm, mask_hbm,
          idx_v, px_v, py_v, pz_v, dist_v, sw_v, mask_v, sem_d, sem_v):
    wid = lax.axis_index("s") * 2 + lax.axis_index("c")
    ones_i = jnp.ones((16,), jnp.int32)
    zeros_i = jnp.zeros((16,), jnp.int32)

    def chunk_body(j, carry):
        c = wid + _NW * j

        @pl.when(c < _NCHUNKS)
        def _():
            base = c * _K
            pltpu.sync_copy(idx_hbm.at[pl.ds(base, _K)], idx_v)
            cp_d = pltpu.async_copy(dist_hbm.at[idx_v], dist_v, sem_d)
            cp_x = pltpu.async_copy(vx_hbm.at[idx_v], px_v, sem_v)
            cp_y = pltpu.async_copy(vy_hbm.at[idx_v], py_v, sem_v)
            cp_z = pltpu.async_copy(vz_hbm.at[idx_v], pz_v, sem_v)
            cp_d.wait()

            def group(g, carry):
                o = g * 16
                d = dist_v[pl.ds(o, 16)]
                m = d < _CUTOFF
                z = d * (math.pi / _CUTOFF) - (math.pi / 2.0)
                z2 = z * z
                s = z * (1.0 + z2 * (_S3 + z2 * (_S5 + z2 * (_S7 + z2 * _S9))))
                sw_v[pl.ds(o, 16)] = jnp.where(m, 0.5 - 0.5 * s, 0.0)
                mask_v[pl.ds(o, 16)] = jnp.where(m, ones_i, zeros_i)
                return carry

            lax.fori_loop(0, _K // 16, group, 0, unroll=4)

            pltpu.sync_copy(dist_v, distf_hbm.at[pl.ds(base, _K)])
            pltpu.sync_copy(sw_v, sw_hbm.at[pl.ds(base, _K)])
            pltpu.sync_copy(mask_v, mask_hbm.at[pl.ds(base, _K)])
            cp_x.wait()
            cp_y.wait()
            cp_z.wait()
            pltpu.sync_copy(px_v, ox_hbm.at[pl.ds(base, _K)])
            pltpu.sync_copy(py_v, oy_hbm.at[pl.ds(base, _K)])
            pltpu.sync_copy(pz_v, oz_hbm.at[pl.ds(base, _K)])

        return carry

    lax.fori_loop(0, _JMAX, chunk_body, 0)


@jax.jit
def _run(vec, distances, filter_indices):
    mesh = plsc.VectorSubcoreMesh(core_axis_name="c", subcore_axis_name="s")
    f32 = jnp.float32
    fn = pl.kernel(
        _body,
        out_type=[
            jax.ShapeDtypeStruct((_E_FILTER,), f32),
            jax.ShapeDtypeStruct((_E_FILTER,), f32),
            jax.ShapeDtypeStruct((_E_FILTER,), f32),
            jax.ShapeDtypeStruct((_E_FILTER,), f32),
            jax.ShapeDtypeStruct((_E_FILTER,), f32),
            jax.ShapeDtypeStruct((_E_FILTER,), jnp.int32),
        ],
        mesh=mesh,
        scratch_types=[
            pltpu.VMEM((_K,), jnp.int32),
            pltpu.VMEM((_K,), f32),
            pltpu.VMEM((_K,), f32),
            pltpu.VMEM((_K,), f32),
            pltpu.VMEM((_K,), f32),
            pltpu.VMEM((_K,), f32),
            pltpu.VMEM((_K,), jnp.int32),
            pltpu.SemaphoreType.DMA,
            pltpu.SemaphoreType.DMA,
        ],
    )
    ox, oy, oz, dist_f, switch, mask_i32 = fn(
        vec[:, 0], vec[:, 1], vec[:, 2], distances, filter_indices)
    vec_f = jnp.stack([ox, oy, oz], axis=1)
    return vec_f, dist_f, switch, mask_i32.astype(jnp.bool_)


def kernel(vec, distances, filter_indices):
    return _run(vec, distances, filter_indices)


# speedup vs baseline: 1.1971x; 1.0346x over previous
import math

import jax
import jax.numpy as jnp
from jax import lax
from jax.experimental import pallas as pl
from jax.experimental.pallas import tpu as pltpu
from jax.experimental.pallas import tpu_sc as plsc

_CUTOFF = 0.5
_E_PARENT = 6_400_000
_E_FILTER = 3_200_000

_K = 5120
_NCHUNKS = _E_FILTER // _K
_NW = 32
_JMAX = -(-_NCHUNKS // _NW)
_JPAIRS = (_JMAX + 1) // 2

_S3 = -1.0 / 6.0
_S5 = 1.0 / 120.0
_S7 = -1.0 / 5040.0
_S9 = 1.0 / 362880.0


def _body(vx_hbm, vy_hbm, vz_hbm, dist_hbm, idx_hbm,
          ox_hbm, oy_hbm, oz_hbm, distf_hbm, sw_hbm, mask_hbm,
          idx_v0, px_v0, py_v0, pz_v0, dist_v0, sw_v0, mask_v0,
          idx_v1, px_v1, py_v1, pz_v1, dist_v1, sw_v1, mask_v1,
          sem_d0, sem_v0, sem_d1, sem_v1):
    wid = lax.axis_index("s") * 2 + lax.axis_index("c")
    ones_i = jnp.ones((16,), jnp.int32)
    zeros_i = jnp.zeros((16,), jnp.int32)

    sets = (
        (idx_v0, px_v0, py_v0, pz_v0, dist_v0, sw_v0, mask_v0, sem_d0, sem_v0),
        (idx_v1, px_v1, py_v1, pz_v1, dist_v1, sw_v1, mask_v1, sem_d1, sem_v1),
    )

    def fire(j, s):
        idx_v, px_v, py_v, pz_v, dist_v, _sw, _mk, sem_d, sem_v = s
        c = wid + _NW * j

        @pl.when(c < _NCHUNKS)
        def _():
            base = c * _K
            pltpu.sync_copy(idx_hbm.at[pl.ds(base, _K)], idx_v)
            pltpu.async_copy(dist_hbm.at[idx_v], dist_v, sem_d)
            pltpu.async_copy(vx_hbm.at[idx_v], px_v, sem_v)
            pltpu.async_copy(vy_hbm.at[idx_v], py_v, sem_v)
            pltpu.async_copy(vz_hbm.at[idx_v], pz_v, sem_v)

    def finish(j, s):
        idx_v, px_v, py_v, pz_v, dist_v, sw_v, mask_v, sem_d, sem_v = s
        c = wid + _NW * j

        @pl.when(c < _NCHUNKS)
        def _():
            base = c * _K
            pltpu.make_async_copy(dist_hbm.at[idx_v], dist_v, sem_d).wait()

            def group(g, carry):
                o = g * 16
                d = dist_v[pl.ds(o, 16)]
                m = d < _CUTOFF
                z = d * (math.pi / _CUTOFF) - (math.pi / 2.0)
                z2 = z * z
                s_ = z * (1.0 + z2 * (_S3 + z2 * (_S5 + z2 * (_S7 + z2 * _S9))))
                sw_v[pl.ds(o, 16)] = jnp.where(m, 0.5 - 0.5 * s_, 0.0)
                mask_v[pl.ds(o, 16)] = jnp.where(m, ones_i, zeros_i)
                return carry

            lax.fori_loop(0, _K // 16, group, 0, unroll=4)

            pltpu.sync_copy(dist_v, distf_hbm.at[pl.ds(base, _K)])
            pltpu.sync_copy(sw_v, sw_hbm.at[pl.ds(base, _K)])
            pltpu.sync_copy(mask_v, mask_hbm.at[pl.ds(base, _K)])
            pltpu.make_async_copy(vx_hbm.at[idx_v], px_v, sem_v).wait()
            pltpu.make_async_copy(vy_hbm.at[idx_v], py_v, sem_v).wait()
            pltpu.make_async_copy(vz_hbm.at[idx_v], pz_v, sem_v).wait()
            pltpu.sync_copy(px_v, ox_hbm.at[pl.ds(base, _K)])
            pltpu.sync_copy(py_v, oy_hbm.at[pl.ds(base, _K)])
            pltpu.sync_copy(pz_v, oz_hbm.at[pl.ds(base, _K)])

    fire(0, sets[0])

    def pair_body(jj, carry):
        j0 = 2 * jj
        fire(j0 + 1, sets[1])
        finish(j0, sets[0])
        fire(j0 + 2, sets[0])
        finish(j0 + 1, sets[1])
        return carry

    lax.fori_loop(0, _JPAIRS, pair_body, 0)


@jax.jit
def _run(vec, distances, filter_indices):
    mesh = plsc.VectorSubcoreMesh(core_axis_name="c", subcore_axis_name="s")
    f32 = jnp.float32
    set_scratch = [
        pltpu.VMEM((_K,), jnp.int32),
        pltpu.VMEM((_K,), f32),
        pltpu.VMEM((_K,), f32),
        pltpu.VMEM((_K,), f32),
        pltpu.VMEM((_K,), f32),
        pltpu.VMEM((_K,), f32),
        pltpu.VMEM((_K,), jnp.int32),
    ]
    fn = pl.kernel(
        _body,
        out_type=[
            jax.ShapeDtypeStruct((_E_FILTER,), f32),
            jax.ShapeDtypeStruct((_E_FILTER,), f32),
            jax.ShapeDtypeStruct((_E_FILTER,), f32),
            jax.ShapeDtypeStruct((_E_FILTER,), f32),
            jax.ShapeDtypeStruct((_E_FILTER,), f32),
            jax.ShapeDtypeStruct((_E_FILTER,), jnp.int32),
        ],
        mesh=mesh,
        scratch_types=set_scratch + set_scratch + [
            pltpu.SemaphoreType.DMA,
            pltpu.SemaphoreType.DMA,
            pltpu.SemaphoreType.DMA,
            pltpu.SemaphoreType.DMA,
        ],
    )
    ox, oy, oz, dist_f, switch, mask_i32 = fn(
        vec[:, 0], vec[:, 1], vec[:, 2], distances, filter_indices)
    vec_f = jnp.stack([ox, oy, oz], axis=1)
    return vec_f, dist_f, switch, mask_i32.astype(jnp.bool_)


def kernel(vec, distances, filter_indices):
    return _run(vec, distances, filter_indices)


# speedup vs baseline: 1.2091x; 1.0101x over previous
import math

import jax
import jax.numpy as jnp
from jax import lax
from jax.experimental import pallas as pl
from jax.experimental.pallas import tpu as pltpu
from jax.experimental.pallas import tpu_sc as plsc

_CUTOFF = 0.5
_E_PARENT = 6_400_000
_E_FILTER = 3_200_000

_K = 2560
_NCHUNKS = _E_FILTER // _K
_NW = 32
_JMAX = -(-_NCHUNKS // _NW)
_JPAIRS = (_JMAX + 1) // 2

_S3 = -1.0 / 6.0
_S5 = 1.0 / 120.0
_S7 = -1.0 / 5040.0
_S9 = 1.0 / 362880.0


def _body(vx_hbm, vy_hbm, vz_hbm, dist_hbm, idx_hbm,
          ox_hbm, oy_hbm, oz_hbm, distf_hbm, sw_hbm, mask_hbm,
          idx_v0, px_v0, py_v0, pz_v0, dist_v0, sw_v0, mask_v0,
          idx_v1, px_v1, py_v1, pz_v1, dist_v1, sw_v1, mask_v1,
          sem_d0, sem_v0, sem_d1, sem_v1):
    wid = lax.axis_index("s") * 2 + lax.axis_index("c")
    ones_i = jnp.ones((16,), jnp.int32)
    zeros_i = jnp.zeros((16,), jnp.int32)

    sets = (
        (idx_v0, px_v0, py_v0, pz_v0, dist_v0, sw_v0, mask_v0, sem_d0, sem_v0),
        (idx_v1, px_v1, py_v1, pz_v1, dist_v1, sw_v1, mask_v1, sem_d1, sem_v1),
    )

    def fire(j, s):
        idx_v, px_v, py_v, pz_v, dist_v, _sw, _mk, sem_d, sem_v = s
        c = wid + _NW * j

        @pl.when(c < _NCHUNKS)
        def _():
            base = c * _K
            pltpu.sync_copy(idx_hbm.at[pl.ds(base, _K)], idx_v)
            pltpu.async_copy(dist_hbm.at[idx_v], dist_v, sem_d)
            pltpu.async_copy(vx_hbm.at[idx_v], px_v, sem_v)
            pltpu.async_copy(vy_hbm.at[idx_v], py_v, sem_v)
            pltpu.async_copy(vz_hbm.at[idx_v], pz_v, sem_v)

    def finish(j, s):
        idx_v, px_v, py_v, pz_v, dist_v, sw_v, mask_v, sem_d, sem_v = s
        c = wid + _NW * j

        @pl.when(c < _NCHUNKS)
        def _():
            base = c * _K
            pltpu.make_async_copy(dist_hbm.at[idx_v], dist_v, sem_d).wait()

            def group(g, carry):
                o = g * 16
                d = dist_v[pl.ds(o, 16)]
                m = d < _CUTOFF
                z = d * (math.pi / _CUTOFF) - (math.pi / 2.0)
                z2 = z * z
                s_ = z * (1.0 + z2 * (_S3 + z2 * (_S5 + z2 * (_S7 + z2 * _S9))))
                sw_v[pl.ds(o, 16)] = jnp.where(m, 0.5 - 0.5 * s_, 0.0)
                mask_v[pl.ds(o, 16)] = jnp.where(m, ones_i, zeros_i)
                return carry

            lax.fori_loop(0, _K // 16, group, 0, unroll=4)

            pltpu.sync_copy(dist_v, distf_hbm.at[pl.ds(base, _K)])
            pltpu.sync_copy(sw_v, sw_hbm.at[pl.ds(base, _K)])
            pltpu.sync_copy(mask_v, mask_hbm.at[pl.ds(base, _K)])
            pltpu.make_async_copy(vx_hbm.at[idx_v], px_v, sem_v).wait()
            pltpu.make_async_copy(vy_hbm.at[idx_v], py_v, sem_v).wait()
            pltpu.make_async_copy(vz_hbm.at[idx_v], pz_v, sem_v).wait()
            pltpu.sync_copy(px_v, ox_hbm.at[pl.ds(base, _K)])
            pltpu.sync_copy(py_v, oy_hbm.at[pl.ds(base, _K)])
            pltpu.sync_copy(pz_v, oz_hbm.at[pl.ds(base, _K)])

    fire(0, sets[0])

    def pair_body(jj, carry):
        j0 = 2 * jj
        fire(j0 + 1, sets[1])
        finish(j0, sets[0])
        fire(j0 + 2, sets[0])
        finish(j0 + 1, sets[1])
        return carry

    lax.fori_loop(0, _JPAIRS, pair_body, 0)


@jax.jit
def _run(vec, distances, filter_indices):
    mesh = plsc.VectorSubcoreMesh(core_axis_name="c", subcore_axis_name="s")
    f32 = jnp.float32
    set_scratch = [
        pltpu.VMEM((_K,), jnp.int32),
        pltpu.VMEM((_K,), f32),
        pltpu.VMEM((_K,), f32),
        pltpu.VMEM((_K,), f32),
        pltpu.VMEM((_K,), f32),
        pltpu.VMEM((_K,), f32),
        pltpu.VMEM((_K,), jnp.int32),
    ]
    fn = pl.kernel(
        _body,
        out_type=[
            jax.ShapeDtypeStruct((_E_FILTER,), f32),
            jax.ShapeDtypeStruct((_E_FILTER,), f32),
            jax.ShapeDtypeStruct((_E_FILTER,), f32),
            jax.ShapeDtypeStruct((_E_FILTER,), f32),
            jax.ShapeDtypeStruct((_E_FILTER,), f32),
            jax.ShapeDtypeStruct((_E_FILTER,), jnp.int32),
        ],
        mesh=mesh,
        scratch_types=set_scratch + set_scratch + [
            pltpu.SemaphoreType.DMA,
            pltpu.SemaphoreType.DMA,
            pltpu.SemaphoreType.DMA,
            pltpu.SemaphoreType.DMA,
        ],
    )
    ox, oy, oz, dist_f, switch, mask_i32 = fn(
        vec[:, 0], vec[:, 1], vec[:, 2], distances, filter_indices)
    vec_f = jnp.stack([ox, oy, oz], axis=1)
    return vec_f, dist_f, switch, mask_i32.astype(jnp.bool_)


def kernel(vec, distances, filter_indices):
    return _run(vec, distances, filter_indices)
